# baseline (speedup 1.0000x reference)
import functools

import jax
import jax.numpy as jnp
import numpy as np
from jax import lax
from jax.experimental import pallas as pl
from jax.experimental.pallas import tpu as pltpu
from jax.experimental.pallas import tpu_sc as plsc

N = 10000
E = 160000
EP = 163840
D = 256
DH = 128
NC = 2
NS = 16
LANES = 16

NPAD = 10240
ZD = NPAD // NS

KD = 40
CPTA = 128
EPT_A = EP // NC // NS

K = 80
EPT_M = EP // NS
CPTC = EPT_M // K
NBUF = 4
RPT = NPAD // NS
ZR = 128

_mesh = plsc.VectorSubcoreMesh(
    core_axis_name="c", subcore_axis_name="s", num_cores=NC, num_subcores=NS)
_sc_params = pltpu.CompilerParams(needs_layout_passes=False)


@functools.partial(
    pl.kernel,
    out_type=(jax.ShapeDtypeStruct((NPAD,), jnp.float32),
              jax.ShapeDtypeStruct((NPAD,), jnp.float32)),
    mesh=_mesh,
    scratch_types=[
        pltpu.VMEM((CPTA, KD), jnp.int32),
        pltpu.VMEM((CPTA, KD), jnp.float32),
        pltpu.VMEM((ZD,), jnp.float32),
        pltpu.VMEM_SHARED((NPAD,), jnp.float32),
        pltpu.SemaphoreType.DMA,
    ],
    compiler_params=_sc_params)
def _deg_kernel(dst2d_hbm, ew2d_hbm, dp0_hbm, dp1_hbm, didx, ewb, zbuf,
                deg_sh, sem):
    cid = lax.axis_index("c")
    sid = lax.axis_index("s")
    zero16 = jnp.zeros((LANES,), jnp.float32)
    for i in range(ZD // LANES):
        zbuf[pl.ds(i * LANES, LANES)] = zero16
    pltpu.sync_copy(zbuf, deg_sh.at[pl.ds(sid * ZD, ZD)])

    row0 = (cid * NS + sid) * CPTA
    pltpu.sync_copy(dst2d_hbm.at[pl.ds(row0, CPTA)], didx)
    pltpu.sync_copy(ew2d_hbm.at[pl.ds(row0, CPTA)], ewb)
    plsc.subcore_barrier()

    def step(s, carry):
        descs = []
        for b in range(8):
            c = s * 8 + b
            descs.append(pltpu.async_copy(
                ewb.at[c], deg_sh.at[didx.at[c]], sem, add=True))
        for d in descs:
            d.wait()
        return carry

    lax.fori_loop(0, CPTA // 8, step, 0)
    plsc.subcore_barrier()

    @pl.when(cid == 0)
    def _():
        pltpu.sync_copy(deg_sh.at[pl.ds(sid * ZD, ZD)],
                        dp0_hbm.at[pl.ds(sid * ZD, ZD)])

    @pl.when(cid == 1)
    def _():
        pltpu.sync_copy(deg_sh.at[pl.ds(sid * ZD, ZD)],
                        dp1_hbm.at[pl.ds(sid * ZD, ZD)])


_RB = 1000


def _linear_body(x_ref, w_ref, dp0_ref, dp1_ref, g_ref, dis_ref):
    deg = dp0_ref[...] + dp1_ref[...] + 1.0
    dis = jnp.where(deg > 0, 1.0 / jnp.sqrt(deg), 0.0)
    h = jnp.dot(x_ref[...], w_ref[...], preferred_element_type=jnp.float32)
    g = h * dis
    g_ref[0] = g[:, :DH]
    g_ref[1] = g[:, DH:]
    dis_ref[...] = dis


_linear_kernel = pl.pallas_call(
    _linear_body,
    grid=(N // _RB,),
    in_specs=[
        pl.BlockSpec((_RB, D), lambda i: (i, 0)),
        pl.BlockSpec((D, D), lambda i: (0, 0)),
        pl.BlockSpec((_RB, 1), lambda i: (i, 0)),
        pl.BlockSpec((_RB, 1), lambda i: (i, 0)),
    ],
    out_specs=[
        pl.BlockSpec((NC, _RB, DH), lambda i: (0, i, 0)),
        pl.BlockSpec((_RB, 1), lambda i: (i, 0)),
    ],
    out_shape=[
        jax.ShapeDtypeStruct((NC, N, DH), jnp.float32),
        jax.ShapeDtypeStruct((N, 1), jnp.float32),
    ],
)


CK = NBUF * K
NSTEPS = CPTC // NBUF


@functools.partial(
    pl.kernel,
    out_type=jax.ShapeDtypeStruct((NC * NPAD, DH), jnp.float32),
    mesh=_mesh,
    scratch_types=[
        pltpu.VMEM((CK,), jnp.int32),
        pltpu.VMEM((CK,), jnp.int32),
        pltpu.VMEM((CK,), jnp.float32),
        pltpu.VMEM((CK,), jnp.float32),
        pltpu.VMEM((2 * NBUF, K), jnp.int32),
        pltpu.VMEM((K, DH), jnp.float32),
        pltpu.VMEM((K, DH), jnp.float32),
        pltpu.VMEM((K, DH), jnp.float32),
        pltpu.VMEM((K, DH), jnp.float32),
        pltpu.VMEM_SHARED((NPAD, DH), jnp.float32),
        pltpu.SemaphoreType.DMA,
        pltpu.SemaphoreType.DMA,
        pltpu.SemaphoreType.DMA,
        pltpu.SemaphoreType.DMA,
        pltpu.SemaphoreType.DMA,
        pltpu.SemaphoreType.DMA,
    ],
    compiler_params=_sc_params)
def _msg_kernel(g_hbm, src_hbm, dst2d_hbm, ew_hbm, acc_hbm,
                srcs0, srcs1, ews0, ews1, didx,
                rows0, rows1, rows2, rows3, acc_sh,
                sem0, sem1, sem2, sem3, psem0, psem1):
    cid = lax.axis_index("c")
    sid = lax.axis_index("s")
    rows_list = (rows0, rows1, rows2, rows3)
    sems = (sem0, sem1, sem2, sem3)
    srcs = (srcs0, srcs1)
    ews = (ews0, ews1)
    psems = (psem0, psem1)
    zero16 = jnp.zeros((LANES,), jnp.float32)

    def fire_loads(s, p):
        e0 = sid * EPT_M + s * CK
        pltpu.async_copy(src_hbm.at[pl.ds(e0, CK)], srcs[p], psems[p])
        pltpu.async_copy(ew_hbm.at[pl.ds(e0, CK)], ews[p], psems[p])

    fire_loads(0, 0)

    def zrow(r, carry):
        for b in range(NBUF):
            for j in range(DH // LANES):
                rows_list[b][r, pl.ds(j * LANES, LANES)] = zero16
        return carry

    lax.fori_loop(0, K, zrow, 0)
    for rnd in range(RPT // (NBUF * K)):
        for b in range(NBUF):
            pltpu.sync_copy(
                rows_list[b],
                acc_sh.at[pl.ds(sid * RPT + (rnd * NBUF + b) * K, K)])
    plsc.subcore_barrier()

    row_off = cid * N

    def outer(s2, carry):
        pltpu.sync_copy(dst2d_hbm.at[pl.ds(sid * CPTC + s2 * 2 * NBUF,
                                           2 * NBUF)], didx)
        for p in (0, 1):
            s = 2 * s2 + p

            @pl.when(s < NSTEPS - 1)
            def _():
                fire_loads(s + 1, 1 - p)

            for _ in range(2):
                pltpu.make_async_copy(src_hbm.at[pl.ds(0, CK)], srcs[p],
                                      psems[p]).wait()
            srcp = srcs[p]
            ewp = ews[p]

            @plsc.parallel_loop(0, CK, LANES, unroll=4)
            def _adj(i):
                sl = pl.ds(i, LANES)
                srcp[sl] = srcp[sl] + row_off

            gdescs = []
            for b in range(NBUF):
                gdescs.append(pltpu.async_copy(
                    g_hbm.at[srcp.at[pl.ds(b * K, K)]], rows_list[b],
                    sems[b]))
            sdescs = []
            for b in range(NBUF):
                gdescs[b].wait()
                rows = rows_list[b]

                @plsc.parallel_loop(0, K, 1, unroll=4)
                def _scale(e):
                    sc = plsc.load_gather(
                        ewp, [jnp.full((LANES,), b * K + e, jnp.int32)])
                    for j in range(DH // LANES):
                        sl = pl.ds(j * LANES, LANES)
                        rows[e, sl] = rows[e, sl] * sc

                sdescs.append(pltpu.async_copy(
                    rows, acc_sh.at[didx.at[p * NBUF + b]], sems[b],
                    add=True))
            for b in range(NBUF):
                sdescs[b].wait()
        return carry

    lax.fori_loop(0, NSTEPS // 2, outer, 0)
    plsc.subcore_barrier()
    for kblk in range(RPT // K):
        r0 = sid * RPT + kblk * K
        pltpu.sync_copy(acc_sh.at[pl.ds(r0, K)],
                        acc_hbm.at[pl.ds(cid * NPAD + r0, K)])


def _norm_body(acc_ref, g_ref, dis_ref, b_ref, gnw_ref, gnb_ref, gms_ref,
               out_ref):
    pre = dis_ref[...] * (acc_ref[0, :N, :] + g_ref[0]) + b_ref[...]
    mean = jnp.mean(pre, axis=0, keepdims=True)
    xs = pre - gms_ref[...] * mean
    var = jnp.mean(xs * xs, axis=0, keepdims=True)
    o = gnw_ref[...] * xs / jnp.sqrt(var + 1e-5) + gnb_ref[...]
    out_ref[...] = o * 0.5 * (1.0 + lax.erf(o * np.float32(1.0 / np.sqrt(2.0))))


_norm_kernel = pl.pallas_call(
    _norm_body,
    grid=(NC,),
    in_specs=[
        pl.BlockSpec((1, NPAD, DH), lambda i: (i, 0, 0)),
        pl.BlockSpec((1, N, DH), lambda i: (i, 0, 0)),
        pl.BlockSpec((N, 1), lambda i: (0, 0)),
        pl.BlockSpec((1, DH), lambda i: (0, i)),
        pl.BlockSpec((1, DH), lambda i: (0, i)),
        pl.BlockSpec((1, DH), lambda i: (0, i)),
        pl.BlockSpec((1, DH), lambda i: (0, i)),
    ],
    out_specs=pl.BlockSpec((N, DH), lambda i: (0, i)),
    out_shape=jax.ShapeDtypeStruct((N, D), jnp.float32),
)


def kernel(x, edge_index, edge_weight, W, b, gn_weight, gn_bias,
           gn_mean_scale):
    src = edge_index[0].astype(jnp.int32)
    dst = edge_index[1].astype(jnp.int32)
    ew = edge_weight.astype(jnp.float32)

    npd = EP - E
    pad_i = jnp.arange(npd, dtype=jnp.int32)
    src_p = jnp.concatenate([src, pad_i % N])
    dst_p = jnp.concatenate([dst, N + pad_i % (NPAD - N)])
    ew_p = jnp.concatenate([ew, jnp.zeros((npd,), jnp.float32)])
    dst2d_a = dst_p.reshape(EP // KD, KD)
    ew2d_a = ew_p.reshape(EP // KD, KD)
    dst2d_c = dst_p.reshape(EP // K, K)

    dp0, dp1 = _deg_kernel(dst2d_a, ew2d_a)
    dp0c = dp0[:N].reshape(N, 1)
    dp1c = dp1[:N].reshape(N, 1)

    g_st, dis = _linear_kernel(x, W, dp0c, dp1c)
    g_flat = g_st.reshape(NC * N, DH)

    acc_flat = _msg_kernel(g_flat, src_p, dst2d_c, ew_p)
    acc_st = acc_flat.reshape(NC, NPAD, DH)

    return _norm_kernel(acc_st, g_st, dis, b.reshape(1, D),
                        gn_weight.reshape(1, D), gn_bias.reshape(1, D),
                        gn_mean_scale.reshape(1, D))


# speedup vs baseline: 1.0220x; 1.0220x over previous
import functools

import jax
import jax.numpy as jnp
import numpy as np
from jax import lax
from jax.experimental import pallas as pl
from jax.experimental.pallas import tpu as pltpu
from jax.experimental.pallas import tpu_sc as plsc

N = 10000
E = 160000
EP = 163840
D = 256
DH = 128
NC = 2
NS = 16
LANES = 16

NPAD = 10240
ZD = NPAD // NS

KD = 40
CPTA = 128
EPT_A = EP // NC // NS

K = 80
EPT_M = EP // NS
CPTC = EPT_M // K
NBUF = 4
RPT = NPAD // NS
ZR = 128

_mesh = plsc.VectorSubcoreMesh(
    core_axis_name="c", subcore_axis_name="s", num_cores=NC, num_subcores=NS)
_sc_params = pltpu.CompilerParams(needs_layout_passes=False)


@functools.partial(
    pl.kernel,
    out_type=(jax.ShapeDtypeStruct((NPAD,), jnp.float32),
              jax.ShapeDtypeStruct((NPAD,), jnp.float32)),
    mesh=_mesh,
    scratch_types=[
        pltpu.VMEM((CPTA, KD), jnp.int32),
        pltpu.VMEM((CPTA, KD), jnp.float32),
        pltpu.VMEM((ZD,), jnp.float32),
        pltpu.VMEM_SHARED((NPAD,), jnp.float32),
        pltpu.SemaphoreType.DMA,
    ],
    compiler_params=_sc_params)
def _deg_kernel(dst2d_hbm, ew2d_hbm, dp0_hbm, dp1_hbm, didx, ewb, zbuf,
                deg_sh, sem):
    cid = lax.axis_index("c")
    sid = lax.axis_index("s")
    zero16 = jnp.zeros((LANES,), jnp.float32)
    for i in range(ZD // LANES):
        zbuf[pl.ds(i * LANES, LANES)] = zero16
    pltpu.sync_copy(zbuf, deg_sh.at[pl.ds(sid * ZD, ZD)])

    row0 = (cid * NS + sid) * CPTA
    pltpu.sync_copy(dst2d_hbm.at[pl.ds(row0, CPTA)], didx)
    pltpu.sync_copy(ew2d_hbm.at[pl.ds(row0, CPTA)], ewb)
    plsc.subcore_barrier()

    def step(s, carry):
        descs = []
        for b in range(8):
            c = s * 8 + b
            descs.append(pltpu.async_copy(
                ewb.at[c], deg_sh.at[didx.at[c]], sem, add=True))
        for d in descs:
            d.wait()
        return carry

    lax.fori_loop(0, CPTA // 8, step, 0)
    plsc.subcore_barrier()

    @pl.when(cid == 0)
    def _():
        pltpu.sync_copy(deg_sh.at[pl.ds(sid * ZD, ZD)],
                        dp0_hbm.at[pl.ds(sid * ZD, ZD)])

    @pl.when(cid == 1)
    def _():
        pltpu.sync_copy(deg_sh.at[pl.ds(sid * ZD, ZD)],
                        dp1_hbm.at[pl.ds(sid * ZD, ZD)])


_RB = 1000


def _linear_body(x_ref, w_ref, dp0_ref, dp1_ref, g_ref, dis_ref):
    deg = dp0_ref[...] + dp1_ref[...] + 1.0
    dis = jnp.where(deg > 0, 1.0 / jnp.sqrt(deg), 0.0)
    h = jnp.dot(x_ref[...], w_ref[...], preferred_element_type=jnp.float32)
    g = h * dis
    g_ref[0] = g[:, :DH]
    g_ref[1] = g[:, DH:]
    dis_ref[...] = dis


_linear_kernel = pl.pallas_call(
    _linear_body,
    grid=(N // _RB,),
    in_specs=[
        pl.BlockSpec((_RB, D), lambda i: (i, 0)),
        pl.BlockSpec((D, D), lambda i: (0, 0)),
        pl.BlockSpec((_RB, 1), lambda i: (i, 0)),
        pl.BlockSpec((_RB, 1), lambda i: (i, 0)),
    ],
    out_specs=[
        pl.BlockSpec((NC, _RB, DH), lambda i: (0, i, 0)),
        pl.BlockSpec((_RB, 1), lambda i: (i, 0)),
    ],
    out_shape=[
        jax.ShapeDtypeStruct((NC, N, DH), jnp.float32),
        jax.ShapeDtypeStruct((N, 1), jnp.float32),
    ],
)


CK = NBUF * K
NSTEPS = CPTC // NBUF


@functools.partial(
    pl.kernel,
    out_type=jax.ShapeDtypeStruct((NC * NPAD, DH), jnp.float32),
    mesh=_mesh,
    scratch_types=[
        pltpu.VMEM((CK,), jnp.int32),
        pltpu.VMEM((CK,), jnp.int32),
        pltpu.VMEM((CK,), jnp.float32),
        pltpu.VMEM((CK,), jnp.float32),
        pltpu.VMEM((2 * NBUF, K), jnp.int32),
        pltpu.VMEM((2 * NBUF, K), jnp.int32),
        pltpu.VMEM((K, DH), jnp.float32),
        pltpu.VMEM((K, DH), jnp.float32),
        pltpu.VMEM((K, DH), jnp.float32),
        pltpu.VMEM((K, DH), jnp.float32),
        pltpu.VMEM_SHARED((NPAD, DH), jnp.float32),
        pltpu.SemaphoreType.DMA,
        pltpu.SemaphoreType.DMA,
        pltpu.SemaphoreType.DMA,
        pltpu.SemaphoreType.DMA,
        pltpu.SemaphoreType.DMA,
        pltpu.SemaphoreType.DMA,
        pltpu.SemaphoreType.DMA,
        pltpu.SemaphoreType.DMA,
    ],
    compiler_params=_sc_params)
def _msg_kernel(g_hbm, src_hbm, dst2d_hbm, ew_hbm, acc_hbm,
                srcs0, srcs1, ews0, ews1, didx0, didx1,
                rows0, rows1, rows2, rows3, acc_sh,
                sem0, sem1, sem2, sem3, psem0, psem1, dsem0, dsem1):
    cid = lax.axis_index("c")
    sid = lax.axis_index("s")
    rows_list = (rows0, rows1, rows2, rows3)
    sems = (sem0, sem1, sem2, sem3)
    srcs = (srcs0, srcs1)
    ews = (ews0, ews1)
    psems = (psem0, psem1)
    didxs = (didx0, didx1)
    dsems = (dsem0, dsem1)
    zero16 = jnp.zeros((LANES,), jnp.float32)

    def fire_loads(s, p):
        e0 = sid * EPT_M + s * CK
        pltpu.async_copy(src_hbm.at[pl.ds(e0, CK)], srcs[p], psems[p])
        pltpu.async_copy(ew_hbm.at[pl.ds(e0, CK)], ews[p], psems[p])

    def fire_didx(s2, q):
        pltpu.async_copy(
            dst2d_hbm.at[pl.ds(sid * CPTC + s2 * 2 * NBUF, 2 * NBUF)],
            didxs[q], dsems[q])

    fire_loads(0, 0)
    fire_didx(0, 0)

    def zrow(r, carry):
        for b in range(NBUF):
            for j in range(DH // LANES):
                rows_list[b][r, pl.ds(j * LANES, LANES)] = zero16
        return carry

    lax.fori_loop(0, K, zrow, 0)
    for rnd in range(RPT // (NBUF * K)):
        for b in range(NBUF):
            pltpu.sync_copy(
                rows_list[b],
                acc_sh.at[pl.ds(sid * RPT + (rnd * NBUF + b) * K, K)])
    plsc.subcore_barrier()

    row_off = cid * N

    NSTEPS2 = NSTEPS // 2

    def outer2(s4, carry):
        pending = []
        for q in (0, 1):
            s2 = 2 * s4 + q
            pltpu.make_async_copy(dst2d_hbm.at[pl.ds(0, 2 * NBUF)],
                                  didxs[q], dsems[q]).wait()
            didx = didxs[q]
            for p in (0, 1):
                s = 2 * s2 + p

                @pl.when(s < NSTEPS - 1)
                def _():
                    fire_loads(s + 1, 1 - p)

                for _ in range(2):
                    pltpu.make_async_copy(src_hbm.at[pl.ds(0, CK)],
                                          srcs[p], psems[p]).wait()
                srcp = srcs[p]
                ewp = ews[p]

                @plsc.parallel_loop(0, CK, LANES, unroll=4)
                def _adj(i):
                    sl = pl.ds(i, LANES)
                    srcp[sl] = srcp[sl] + row_off

                for d2 in pending:
                    d2.wait()
                pending = []

                if p == 0:
                    @pl.when(s2 < NSTEPS2 - 1)
                    def _():
                        fire_didx(s2 + 1, 1 - q)

                gdescs = []
                for b in range(NBUF):
                    gdescs.append(pltpu.async_copy(
                        g_hbm.at[srcp.at[pl.ds(b * K, K)]], rows_list[b],
                        sems[b]))
                for b in range(NBUF):
                    gdescs[b].wait()
                    rows = rows_list[b]

                    @plsc.parallel_loop(0, K, 1, unroll=4)
                    def _scale(e):
                        sc = plsc.load_gather(
                            ewp, [jnp.full((LANES,), b * K + e, jnp.int32)])
                        for j in range(DH // LANES):
                            sl = pl.ds(j * LANES, LANES)
                            rows[e, sl] = rows[e, sl] * sc

                    pending.append(pltpu.async_copy(
                        rows, acc_sh.at[didx.at[p * NBUF + b]], sems[b],
                        add=True))
        for d2 in pending:
            d2.wait()
        return carry

    lax.fori_loop(0, NSTEPS2 // 2, outer2, 0)
    plsc.subcore_barrier()
    for kblk in range(RPT // K):
        r0 = sid * RPT + kblk * K
        pltpu.sync_copy(acc_sh.at[pl.ds(r0, K)],
                        acc_hbm.at[pl.ds(cid * NPAD + r0, K)])


def _norm_body(acc_ref, g_ref, dis_ref, b_ref, gnw_ref, gnb_ref, gms_ref,
               out_ref):
    pre = dis_ref[...] * (acc_ref[0, :N, :] + g_ref[0]) + b_ref[...]
    mean = jnp.mean(pre, axis=0, keepdims=True)
    xs = pre - gms_ref[...] * mean
    var = jnp.mean(xs * xs, axis=0, keepdims=True)
    o = gnw_ref[...] * xs / jnp.sqrt(var + 1e-5) + gnb_ref[...]
    out_ref[...] = o * 0.5 * (1.0 + lax.erf(o * np.float32(1.0 / np.sqrt(2.0))))


_norm_kernel = pl.pallas_call(
    _norm_body,
    grid=(NC,),
    in_specs=[
        pl.BlockSpec((1, NPAD, DH), lambda i: (i, 0, 0)),
        pl.BlockSpec((1, N, DH), lambda i: (i, 0, 0)),
        pl.BlockSpec((N, 1), lambda i: (0, 0)),
        pl.BlockSpec((1, DH), lambda i: (0, i)),
        pl.BlockSpec((1, DH), lambda i: (0, i)),
        pl.BlockSpec((1, DH), lambda i: (0, i)),
        pl.BlockSpec((1, DH), lambda i: (0, i)),
    ],
    out_specs=pl.BlockSpec((N, DH), lambda i: (0, i)),
    out_shape=jax.ShapeDtypeStruct((N, D), jnp.float32),
)


def kernel(x, edge_index, edge_weight, W, b, gn_weight, gn_bias,
           gn_mean_scale):
    src = edge_index[0].astype(jnp.int32)
    dst = edge_index[1].astype(jnp.int32)
    ew = edge_weight.astype(jnp.float32)

    npd = EP - E
    pad_i = jnp.arange(npd, dtype=jnp.int32)
    src_p = jnp.concatenate([src, pad_i % N])
    dst_p = jnp.concatenate([dst, N + pad_i % (NPAD - N)])
    ew_p = jnp.concatenate([ew, jnp.zeros((npd,), jnp.float32)])
    dst2d_a = dst_p.reshape(EP // KD, KD)
    ew2d_a = ew_p.reshape(EP // KD, KD)
    dst2d_c = dst_p.reshape(EP // K, K)

    dp0, dp1 = _deg_kernel(dst2d_a, ew2d_a)
    dp0c = dp0[:N].reshape(N, 1)
    dp1c = dp1[:N].reshape(N, 1)

    g_st, dis = _linear_kernel(x, W, dp0c, dp1c)
    g_flat = g_st.reshape(NC * N, DH)

    acc_flat = _msg_kernel(g_flat, src_p, dst2d_c, ew_p)
    acc_st = acc_flat.reshape(NC, NPAD, DH)

    return _norm_kernel(acc_st, g_st, dis, b.reshape(1, D),
                        gn_weight.reshape(1, D), gn_bias.reshape(1, D),
                        gn_mean_scale.reshape(1, D))


# speedup vs baseline: 1.0223x; 1.0003x over previous
import functools

import jax
import jax.numpy as jnp
import numpy as np
from jax import lax
from jax.experimental import pallas as pl
from jax.experimental.pallas import tpu as pltpu
from jax.experimental.pallas import tpu_sc as plsc

N = 10000
E = 160000
EP = 163840
D = 256
DH = 128
NC = 2
NS = 16
LANES = 16

NPAD = 10240
ZD = NPAD // NS

KD = 40
CPTA = 128
EPT_A = EP // NC // NS

K = 80
EPT_M = EP // NS
CPTC = EPT_M // K
NBUF = 4
RPT = NPAD // NS
ZR = 128

_mesh = plsc.VectorSubcoreMesh(
    core_axis_name="c", subcore_axis_name="s", num_cores=NC, num_subcores=NS)
_sc_params = pltpu.CompilerParams(needs_layout_passes=False)


@functools.partial(
    pl.kernel,
    out_type=(jax.ShapeDtypeStruct((NPAD,), jnp.float32),
              jax.ShapeDtypeStruct((NPAD,), jnp.float32)),
    mesh=_mesh,
    scratch_types=[
        pltpu.VMEM((CPTA, KD), jnp.int32),
        pltpu.VMEM((CPTA, KD), jnp.float32),
        pltpu.VMEM((ZD,), jnp.float32),
        pltpu.VMEM_SHARED((NPAD,), jnp.float32),
        pltpu.SemaphoreType.DMA,
    ],
    compiler_params=_sc_params)
def _deg_kernel(dst2d_hbm, ew2d_hbm, dp0_hbm, dp1_hbm, didx, ewb, zbuf,
                deg_sh, sem):
    cid = lax.axis_index("c")
    sid = lax.axis_index("s")
    zero16 = jnp.zeros((LANES,), jnp.float32)
    for i in range(ZD // LANES):
        zbuf[pl.ds(i * LANES, LANES)] = zero16
    pltpu.sync_copy(zbuf, deg_sh.at[pl.ds(sid * ZD, ZD)])

    row0 = (cid * NS + sid) * CPTA
    pltpu.sync_copy(dst2d_hbm.at[pl.ds(row0, CPTA)], didx)
    pltpu.sync_copy(ew2d_hbm.at[pl.ds(row0, CPTA)], ewb)
    plsc.subcore_barrier()

    def step(s, carry):
        descs = []
        for b in range(8):
            c = s * 8 + b
            descs.append(pltpu.async_copy(
                ewb.at[c], deg_sh.at[didx.at[c]], sem, add=True))
        for d in descs:
            d.wait()
        return carry

    lax.fori_loop(0, CPTA // 8, step, 0)
    plsc.subcore_barrier()

    @pl.when(cid == 0)
    def _():
        pltpu.sync_copy(deg_sh.at[pl.ds(sid * ZD, ZD)],
                        dp0_hbm.at[pl.ds(sid * ZD, ZD)])

    @pl.when(cid == 1)
    def _():
        pltpu.sync_copy(deg_sh.at[pl.ds(sid * ZD, ZD)],
                        dp1_hbm.at[pl.ds(sid * ZD, ZD)])


_RB = 1000


def _linear_body(x_ref, w_ref, dp0_ref, dp1_ref, g_ref, dis_ref):
    deg = dp0_ref[...] + dp1_ref[...] + 1.0
    dis = jnp.where(deg > 0, 1.0 / jnp.sqrt(deg), 0.0)
    h = jnp.dot(x_ref[...], w_ref[...], preferred_element_type=jnp.float32)
    g = h * dis
    g_ref[0] = g[:, :DH]
    g_ref[1] = g[:, DH:]
    dis_ref[...] = dis


_linear_kernel = pl.pallas_call(
    _linear_body,
    grid=(N // _RB,),
    in_specs=[
        pl.BlockSpec((_RB, D), lambda i: (i, 0)),
        pl.BlockSpec((D, D), lambda i: (0, 0)),
        pl.BlockSpec((_RB, 1), lambda i: (i, 0)),
        pl.BlockSpec((_RB, 1), lambda i: (i, 0)),
    ],
    out_specs=[
        pl.BlockSpec((NC, _RB, DH), lambda i: (0, i, 0)),
        pl.BlockSpec((_RB, 1), lambda i: (i, 0)),
    ],
    out_shape=[
        jax.ShapeDtypeStruct((NC, N, DH), jnp.float32),
        jax.ShapeDtypeStruct((N, 1), jnp.float32),
    ],
)


CK = NBUF * K
NSTEPS = CPTC // NBUF


@functools.partial(
    pl.kernel,
    out_type=jax.ShapeDtypeStruct((NC * NPAD, DH), jnp.float32),
    mesh=_mesh,
    scratch_types=[
        pltpu.VMEM((CK,), jnp.int32),
        pltpu.VMEM((CK,), jnp.int32),
        pltpu.VMEM((CK,), jnp.float32),
        pltpu.VMEM((CK,), jnp.float32),
        pltpu.VMEM((2 * NBUF, K), jnp.int32),
        pltpu.VMEM((2 * NBUF, K), jnp.int32),
        pltpu.VMEM((K, DH), jnp.float32),
        pltpu.VMEM((K, DH), jnp.float32),
        pltpu.VMEM((K, DH), jnp.float32),
        pltpu.VMEM((K, DH), jnp.float32),
        pltpu.VMEM_SHARED((NPAD, DH), jnp.float32),
        pltpu.SemaphoreType.DMA,
        pltpu.SemaphoreType.DMA,
        pltpu.SemaphoreType.DMA,
        pltpu.SemaphoreType.DMA,
        pltpu.SemaphoreType.DMA,
        pltpu.SemaphoreType.DMA,
        pltpu.SemaphoreType.DMA,
        pltpu.SemaphoreType.DMA,
    ],
    compiler_params=_sc_params)
def _msg_kernel(g_hbm, src_hbm, dst2d_hbm, ew_hbm, acc_hbm,
                srcs0, srcs1, ews0, ews1, didx0, didx1,
                rows0, rows1, rows2, rows3, acc_sh,
                sem0, sem1, sem2, sem3, psem0, psem1, dsem0, dsem1):
    cid = lax.axis_index("c")
    sid = lax.axis_index("s")
    rows_list = (rows0, rows1, rows2, rows3)
    sems = (sem0, sem1, sem2, sem3)
    srcs = (srcs0, srcs1)
    ews = (ews0, ews1)
    psems = (psem0, psem1)
    didxs = (didx0, didx1)
    dsems = (dsem0, dsem1)
    zero16 = jnp.zeros((LANES,), jnp.float32)

    def fire_loads(s, p):
        e0 = sid * EPT_M + s * CK
        pltpu.async_copy(src_hbm.at[pl.ds(e0, CK)], srcs[p], psems[p])
        pltpu.async_copy(ew_hbm.at[pl.ds(e0, CK)], ews[p], psems[p])

    def fire_didx(s2, q):
        pltpu.async_copy(
            dst2d_hbm.at[pl.ds(sid * CPTC + s2 * 2 * NBUF, 2 * NBUF)],
            didxs[q], dsems[q])

    fire_loads(0, 0)
    fire_didx(0, 0)

    def zrow(r, carry):
        for b in range(NBUF):
            for j in range(DH // LANES):
                rows_list[b][r, pl.ds(j * LANES, LANES)] = zero16
        return carry

    lax.fori_loop(0, K, zrow, 0)
    zdescs = []
    for rnd in range(RPT // (NBUF * K)):
        for b in range(NBUF):
            zdescs.append(pltpu.async_copy(
                rows_list[b],
                acc_sh.at[pl.ds(sid * RPT + (rnd * NBUF + b) * K, K)],
                sems[b]))
    for d in zdescs:
        d.wait()
    plsc.subcore_barrier()

    row_off = cid * N

    NSTEPS2 = NSTEPS // 2

    def outer2(s4, carry):
        pending = []
        for q in (0, 1):
            s2 = 2 * s4 + q
            pltpu.make_async_copy(dst2d_hbm.at[pl.ds(0, 2 * NBUF)],
                                  didxs[q], dsems[q]).wait()
            didx = didxs[q]
            for p in (0, 1):
                s = 2 * s2 + p

                @pl.when(s < NSTEPS - 1)
                def _():
                    fire_loads(s + 1, 1 - p)

                for _ in range(2):
                    pltpu.make_async_copy(src_hbm.at[pl.ds(0, CK)],
                                          srcs[p], psems[p]).wait()
                srcp = srcs[p]
                ewp = ews[p]

                @plsc.parallel_loop(0, CK, LANES, unroll=4)
                def _adj(i):
                    sl = pl.ds(i, LANES)
                    srcp[sl] = srcp[sl] + row_off

                for d2 in pending:
                    d2.wait()
                pending = []

                if p == 0:
                    @pl.when(s2 < NSTEPS2 - 1)
                    def _():
                        fire_didx(s2 + 1, 1 - q)

                gdescs = []
                for b in range(NBUF):
                    gdescs.append(pltpu.async_copy(
                        g_hbm.at[srcp.at[pl.ds(b * K, K)]], rows_list[b],
                        sems[b]))
                for b in range(NBUF):
                    gdescs[b].wait()
                    rows = rows_list[b]

                    @plsc.parallel_loop(0, K, 1, unroll=4)
                    def _scale(e):
                        sc = plsc.load_gather(
                            ewp, [jnp.full((LANES,), b * K + e, jnp.int32)])
                        for j in range(DH // LANES):
                            sl = pl.ds(j * LANES, LANES)
                            rows[e, sl] = rows[e, sl] * sc

                    pending.append(pltpu.async_copy(
                        rows, acc_sh.at[didx.at[p * NBUF + b]], sems[b],
                        add=True))
        for d2 in pending:
            d2.wait()
        return carry

    lax.fori_loop(0, NSTEPS2 // 2, outer2, 0)
    plsc.subcore_barrier()
    wdescs = []
    for kblk in range(RPT // K):
        r0 = sid * RPT + kblk * K
        wdescs.append(pltpu.async_copy(
            acc_sh.at[pl.ds(r0, K)],
            acc_hbm.at[pl.ds(cid * NPAD + r0, K)], sems[kblk % NBUF]))
    for d in wdescs:
        d.wait()


def _norm_body(acc_ref, g_ref, dis_ref, b_ref, gnw_ref, gnb_ref, gms_ref,
               out_ref):
    pre = dis_ref[...] * (acc_ref[0, :N, :] + g_ref[0]) + b_ref[...]
    mean = jnp.mean(pre, axis=0, keepdims=True)
    xs = pre - gms_ref[...] * mean
    var = jnp.mean(xs * xs, axis=0, keepdims=True)
    o = gnw_ref[...] * xs / jnp.sqrt(var + 1e-5) + gnb_ref[...]
    out_ref[...] = o * 0.5 * (1.0 + lax.erf(o * np.float32(1.0 / np.sqrt(2.0))))


_norm_kernel = pl.pallas_call(
    _norm_body,
    grid=(NC,),
    in_specs=[
        pl.BlockSpec((1, NPAD, DH), lambda i: (i, 0, 0)),
        pl.BlockSpec((1, N, DH), lambda i: (i, 0, 0)),
        pl.BlockSpec((N, 1), lambda i: (0, 0)),
        pl.BlockSpec((1, DH), lambda i: (0, i)),
        pl.BlockSpec((1, DH), lambda i: (0, i)),
        pl.BlockSpec((1, DH), lambda i: (0, i)),
        pl.BlockSpec((1, DH), lambda i: (0, i)),
    ],
    out_specs=pl.BlockSpec((N, DH), lambda i: (0, i)),
    out_shape=jax.ShapeDtypeStruct((N, D), jnp.float32),
)


def kernel(x, edge_index, edge_weight, W, b, gn_weight, gn_bias,
           gn_mean_scale):
    src = edge_index[0].astype(jnp.int32)
    dst = edge_index[1].astype(jnp.int32)
    ew = edge_weight.astype(jnp.float32)

    npd = EP - E
    pad_i = jnp.arange(npd, dtype=jnp.int32)
    src_p = jnp.concatenate([src, pad_i % N])
    dst_p = jnp.concatenate([dst, N + pad_i % (NPAD - N)])
    ew_p = jnp.concatenate([ew, jnp.zeros((npd,), jnp.float32)])
    dst2d_a = dst_p.reshape(EP // KD, KD)
    ew2d_a = ew_p.reshape(EP // KD, KD)
    dst2d_c = dst_p.reshape(EP // K, K)

    dp0, dp1 = _deg_kernel(dst2d_a, ew2d_a)
    dp0c = dp0[:N].reshape(N, 1)
    dp1c = dp1[:N].reshape(N, 1)

    g_st, dis = _linear_kernel(x, W, dp0c, dp1c)
    g_flat = g_st.reshape(NC * N, DH)

    acc_flat = _msg_kernel(g_flat, src_p, dst2d_c, ew_p)
    acc_st = acc_flat.reshape(NC, NPAD, DH)

    return _norm_kernel(acc_st, g_st, dis, b.reshape(1, D),
                        gn_weight.reshape(1, D), gn_bias.reshape(1, D),
                        gn_mean_scale.reshape(1, D))
